# baseline (speedup 1.0000x reference)
import functools

import jax
import jax.numpy as jnp
from jax import lax
from jax.experimental import pallas as pl
from jax.experimental.pallas import tpu as pltpu
from jax.experimental.pallas import tpu_sc as plsc

_NUM_CORES = 2
_NUM_SUBCORES = 16
_NUM_WORKERS = _NUM_CORES * _NUM_SUBCORES

_CHUNK_B = 8
_CONV_CHUNK = 400


@functools.lru_cache(maxsize=None)
def _make_gather(batch: int, hist: int, n_emb: int, d: int):
    b_per_w = batch // _NUM_WORKERS
    steps = b_per_w // _CHUNK_B
    assert steps * _CHUNK_B == b_per_w
    n_conv_chunks = n_emb // _CONV_CHUNK
    assert n_conv_chunks * _CONV_CHUNK == n_emb
    mesh = plsc.VectorSubcoreMesh(core_axis_name="c", subcore_axis_name="s")

    @functools.partial(
        pl.kernel,
        out_type=jax.ShapeDtypeStruct((batch, hist, d), jnp.float32),
        mesh=mesh,
        compiler_params=pltpu.CompilerParams(use_tc_tiling_on_sc=True),
        scratch_types=[
            pltpu.HBM((n_emb, d), jnp.float32),
            pltpu.VMEM((_CONV_CHUNK, d), jnp.float32),
            pltpu.VMEM((_CHUNK_B, hist), jnp.int32),
            pltpu.VMEM((_CHUNK_B, hist, d), jnp.float32),
            pltpu.SemaphoreType.DMA,
        ],
    )
    def gather(tids_hbm, w_hbm, out_hbm, wconv, cbuf, idx_v, rows_v, gsem):
        sid = lax.axis_index("s")
        cid = lax.axis_index("c")
        wid = sid * _NUM_CORES + cid

        n_mine = (n_conv_chunks - 1 - sid) // _NUM_SUBCORES + 1

        def conv_step(i, carry):
            k = sid + i * _NUM_SUBCORES
            r0 = pl.multiple_of(k * _CONV_CHUNK, 8)
            pltpu.sync_copy(w_hbm.at[pl.ds(r0, _CONV_CHUNK)], cbuf)
            pltpu.sync_copy(cbuf, wconv.at[pl.ds(r0, _CONV_CHUNK)])
            return carry

        lax.fori_loop(0, n_mine, conv_step, 0)
        plsc.subcore_barrier()

        base = wid * b_per_w

        def step(i, carry):
            b0 = base + i * _CHUNK_B
            pltpu.sync_copy(tids_hbm.at[pl.ds(b0, _CHUNK_B)], idx_v)
            handles = []
            for r in range(_CHUNK_B):
                handles.append(
                    pltpu.async_copy(wconv.at[idx_v.at[r]], rows_v.at[r],
                                     gsem))
            for h in handles:
                h.wait()
            pltpu.sync_copy(rows_v, out_hbm.at[pl.ds(b0, _CHUNK_B)])
            return carry

        lax.fori_loop(0, steps, step, 0)

    return gather


def kernel(token_ids, W):
    batch, hist = token_ids.shape
    n_emb, d = W.shape
    return _make_gather(batch, hist, n_emb, d)(token_ids, W)


# speedup vs baseline: 1.5341x; 1.5341x over previous
import functools

import jax
import jax.numpy as jnp
from jax import lax
from jax.experimental import pallas as pl
from jax.experimental.pallas import tpu as pltpu
from jax.experimental.pallas import tpu_sc as plsc

_NUM_CORES = 2
_NUM_SUBCORES = 16
_NUM_WORKERS = _NUM_CORES * _NUM_SUBCORES

_CHUNK_B = 16


@functools.lru_cache(maxsize=None)
def _make_gather(batch: int, hist: int, d: int):
    b_per_w = batch // _NUM_WORKERS
    steps = b_per_w // _CHUNK_B
    assert steps * _CHUNK_B == b_per_w
    mesh = plsc.VectorSubcoreMesh(core_axis_name="c", subcore_axis_name="s")

    @functools.partial(
        pl.kernel,
        out_type=jax.ShapeDtypeStruct((batch, hist, d), jnp.float32),
        mesh=mesh,
        compiler_params=pltpu.CompilerParams(use_tc_tiling_on_sc=False),
        scratch_types=[
            pltpu.VMEM((_CHUNK_B, hist), jnp.int32),
            pltpu.VMEM((_CHUNK_B, hist, d), jnp.float32),
            pltpu.SemaphoreType.DMA,
        ],
    )
    def gather(tids_hbm, w_hbm, out_hbm, idx_v, rows_v, sem):
        wid = lax.axis_index("s") * _NUM_CORES + lax.axis_index("c")
        base = wid * b_per_w

        def step(i, carry):
            b0 = base + i * _CHUNK_B
            pltpu.sync_copy(tids_hbm.at[pl.ds(b0, _CHUNK_B)], idx_v)
            handles = []
            for r in range(_CHUNK_B):
                handles.append(
                    pltpu.async_copy(w_hbm.at[idx_v.at[r]], rows_v.at[r],
                                     sem))
            for h in handles:
                h.wait()
            pltpu.sync_copy(rows_v, out_hbm.at[pl.ds(b0, _CHUNK_B)])
            return carry

        lax.fori_loop(0, steps, step, 0)

    return gather


def kernel(token_ids, W):
    batch, hist = token_ids.shape
    _, d = W.shape
    return _make_gather(batch, hist, d)(token_ids, W)
